# baseline (device time: 211487 ns/iter reference)
import jax
import jax.numpy as jnp
from jax import lax
from jax.experimental import pallas as pl
from jax.experimental.pallas import tpu as pltpu

N_DEV = 8


def kernel(x, Wg, Wu, Wd):
    m, d = x.shape
    h_per = Wg.shape[1]

    def body(x_ref, wg_ref, wu_ref, wd_ref, out_ref, comm_ref, send_sems, recv_sems):
        my = lax.axis_index("i")
        left = lax.rem(my + N_DEV - 1, N_DEV)
        right = lax.rem(my + 1, N_DEV)

        barrier_sem = pltpu.get_barrier_semaphore()
        for nbr in (left, right):
            pl.semaphore_signal(
                barrier_sem, inc=1,
                device_id=(nbr,), device_id_type=pl.DeviceIdType.MESH,
            )
        pl.semaphore_wait(barrier_sem, 2)

        gate = jnp.dot(x_ref[:, :], wg_ref[:, :], preferred_element_type=jnp.float32)
        up = jnp.dot(x_ref[:, :], wu_ref[:, :], preferred_element_type=jnp.float32)
        hidden = gate * (up * jax.nn.sigmoid(up))
        partial = jnp.dot(hidden, wd_ref[:, :], preferred_element_type=jnp.float32)

        comm_ref[0, :, :] = partial
        out_ref[:, :] = partial

        for h in range(N_DEV - 1):
            rdma = pltpu.make_async_remote_copy(
                src_ref=comm_ref.at[h],
                dst_ref=comm_ref.at[h + 1],
                send_sem=send_sems.at[h],
                recv_sem=recv_sems.at[h],
                device_id=(right,),
                device_id_type=pl.DeviceIdType.MESH,
            )
            rdma.start()
            rdma.wait()
            out_ref[:, :] += comm_ref[h + 1, :, :]

    return pl.pallas_call(
        body,
        out_shape=jax.ShapeDtypeStruct((m, d), jnp.float32),
        in_specs=[pl.BlockSpec(memory_space=pltpu.VMEM)] * 4,
        out_specs=pl.BlockSpec(memory_space=pltpu.VMEM),
        scratch_shapes=[
            pltpu.VMEM((N_DEV, m, d), jnp.float32),
            pltpu.SemaphoreType.DMA((N_DEV - 1,)),
            pltpu.SemaphoreType.DMA((N_DEV - 1,)),
        ],
        compiler_params=pltpu.CompilerParams(collective_id=0),
    )(x, Wg, Wu, Wd)


# device time: 74275 ns/iter; 2.8474x vs baseline; 2.8474x over previous
import jax
import jax.numpy as jnp
from jax import lax
from jax.experimental import pallas as pl
from jax.experimental.pallas import tpu as pltpu

N_DEV = 8


def kernel(x, Wg, Wu, Wd):
    m, d = x.shape

    B = m // N_DEV

    def body(x_ref, wg_ref, wu_ref, wd_ref, out_ref, recv_ref, send_sems, recv_sems):
        my = lax.axis_index("i")
        a1 = (my >> 1) & 1
        a2 = (my >> 2) & 1
        a0 = (my & 1) ^ a1
        partners = [my ^ 1, my ^ 3, my ^ 4]

        barrier_sem = pltpu.get_barrier_semaphore()
        for p in partners:
            pl.semaphore_signal(
                barrier_sem, inc=1,
                device_id=(p,), device_id_type=pl.DeviceIdType.MESH,
            )
        pl.semaphore_wait(barrier_sem, 3)

        gate = jnp.dot(x_ref[:, :], wg_ref[:, :], preferred_element_type=jnp.float32)
        up = jnp.dot(x_ref[:, :], wu_ref[:, :], preferred_element_type=jnp.float32)
        hidden = gate * (up * jax.nn.sigmoid(up))
        out_ref[:, :] = jnp.dot(hidden, wd_ref[:, :], preferred_element_type=jnp.float32)

        keep0 = a0 * (4 * B)
        send0 = (1 - a0) * (4 * B)
        keep1 = (4 * a0 + 2 * a1) * B
        send1 = (4 * a0 + 2 * (1 - a1)) * B
        keep2 = (4 * a0 + 2 * a1 + a2) * B
        send2 = (4 * a0 + 2 * a1 + (1 - a2)) * B

        rs = [
            (partners[0], send0, 0, keep0, 4 * B),
            (partners[1], send1, 4 * B, keep1, 2 * B),
            (partners[2], send2, 6 * B, keep2, B),
        ]
        for k, (p, s_off, r_off, k_off, n) in enumerate(rs):
            rdma = pltpu.make_async_remote_copy(
                src_ref=out_ref.at[pl.ds(s_off, n), :],
                dst_ref=recv_ref.at[pl.ds(r_off, n), :],
                send_sem=send_sems.at[k],
                recv_sem=recv_sems.at[k],
                device_id=(p,),
                device_id_type=pl.DeviceIdType.MESH,
            )
            rdma.start()
            rdma.wait()
            out_ref[pl.ds(k_off, n), :] += recv_ref[pl.ds(r_off, n), :]

        ag = [
            (partners[2], keep2, B),
            (partners[1], keep1, 2 * B),
            (partners[0], keep0, 4 * B),
        ]
        for k, (p, off, n) in enumerate(ag):
            rdma = pltpu.make_async_remote_copy(
                src_ref=out_ref.at[pl.ds(off, n), :],
                dst_ref=out_ref.at[pl.ds(off, n), :],
                send_sem=send_sems.at[3 + k],
                recv_sem=recv_sems.at[3 + k],
                device_id=(p,),
                device_id_type=pl.DeviceIdType.MESH,
            )
            rdma.start()
            rdma.wait()

    return pl.pallas_call(
        body,
        out_shape=jax.ShapeDtypeStruct((m, d), jnp.float32),
        in_specs=[pl.BlockSpec(memory_space=pltpu.VMEM)] * 4,
        out_specs=pl.BlockSpec(memory_space=pltpu.VMEM),
        scratch_shapes=[
            pltpu.VMEM((7 * (m // N_DEV), d), jnp.float32),
            pltpu.SemaphoreType.DMA((6,)),
            pltpu.SemaphoreType.DMA((6,)),
        ],
        compiler_params=pltpu.CompilerParams(collective_id=0),
    )(x, Wg, Wu, Wd)


# device time: 54622 ns/iter; 3.8718x vs baseline; 1.3598x over previous
import jax
import jax.numpy as jnp
from jax import lax
from jax.experimental import pallas as pl
from jax.experimental.pallas import tpu as pltpu

N_DEV = 8


def kernel(x, Wg, Wu, Wd):
    m, d = x.shape
    B = m // N_DEV

    def body(x_ref, wg_ref, wu_ref, wd_ref, out_ref,
             rs_recv, rs_send_sems, rs_recv_sems, ag_send_sems, ag_recv_sems):
        my = lax.axis_index("i")
        peers = [lax.rem(my + k, N_DEV) for k in range(1, N_DEV)]

        barrier_sem = pltpu.get_barrier_semaphore()
        for p in peers:
            pl.semaphore_signal(
                barrier_sem, inc=1,
                device_id=(p,), device_id_type=pl.DeviceIdType.MESH,
            )
        pl.semaphore_wait(barrier_sem, N_DEV - 1)

        gate = jnp.dot(x_ref[:, :], wg_ref[:, :], preferred_element_type=jnp.float32)
        up = jnp.dot(x_ref[:, :], wu_ref[:, :], preferred_element_type=jnp.float32)
        hidden = gate * (up * jax.nn.sigmoid(up))
        out_ref[:, :] = jnp.dot(hidden, wd_ref[:, :], preferred_element_type=jnp.float32)

        def rs_desc(p):
            return pltpu.make_async_remote_copy(
                src_ref=out_ref.at[pl.ds(p * B, B), :],
                dst_ref=rs_recv.at[my],
                send_sem=rs_send_sems.at[p],
                recv_sem=rs_recv_sems.at[my],
                device_id=(p,),
                device_id_type=pl.DeviceIdType.MESH,
            )

        def rs_wait_desc(q):
            return pltpu.make_async_remote_copy(
                src_ref=rs_recv.at[q],
                dst_ref=rs_recv.at[q],
                send_sem=rs_recv_sems.at[q],
                recv_sem=rs_recv_sems.at[q],
                device_id=(q,),
                device_id_type=pl.DeviceIdType.MESH,
            )

        for p in peers:
            rs_desc(p).start()

        acc = out_ref[pl.ds(my * B, B), :]
        for q in peers:
            rs_wait_desc(q).wait_recv()
            acc = acc + rs_recv[q]
        out_ref[pl.ds(my * B, B), :] = acc

        def ag_desc(p):
            return pltpu.make_async_remote_copy(
                src_ref=out_ref.at[pl.ds(my * B, B), :],
                dst_ref=out_ref.at[pl.ds(my * B, B), :],
                send_sem=ag_send_sems.at[p],
                recv_sem=ag_recv_sems.at[my],
                device_id=(p,),
                device_id_type=pl.DeviceIdType.MESH,
            )

        def ag_wait_desc(q):
            return pltpu.make_async_remote_copy(
                src_ref=out_ref.at[pl.ds(q * B, B), :],
                dst_ref=out_ref.at[pl.ds(q * B, B), :],
                send_sem=ag_recv_sems.at[q],
                recv_sem=ag_recv_sems.at[q],
                device_id=(q,),
                device_id_type=pl.DeviceIdType.MESH,
            )

        for p in peers:
            ag_desc(p).start()

        for q in peers:
            ag_wait_desc(q).wait_recv()

        for p in peers:
            rs_desc(p).wait_send()
            ag_desc(p).wait_send()

    return pl.pallas_call(
        body,
        out_shape=jax.ShapeDtypeStruct((m, d), jnp.float32),
        in_specs=[pl.BlockSpec(memory_space=pltpu.VMEM)] * 4,
        out_specs=pl.BlockSpec(memory_space=pltpu.VMEM),
        scratch_shapes=[
            pltpu.VMEM((N_DEV, B, d), jnp.float32),
            pltpu.SemaphoreType.DMA((N_DEV,)),
            pltpu.SemaphoreType.DMA((N_DEV,)),
            pltpu.SemaphoreType.DMA((N_DEV,)),
            pltpu.SemaphoreType.DMA((N_DEV,)),
        ],
        compiler_params=pltpu.CompilerParams(collective_id=0),
    )(x, Wg, Wu, Wd)


# device time: 40246 ns/iter; 5.2549x vs baseline; 1.3572x over previous
import jax
import jax.numpy as jnp
from jax import lax
from jax.experimental import pallas as pl
from jax.experimental.pallas import tpu as pltpu

N_DEV = 8


def kernel(x, Wg, Wu, Wd):
    m, d = x.shape
    B = m // N_DEV

    def body(x_ref, wg_ref, wu_ref, wd_ref, out_ref,
             rs_send, rs_recv, ag_send, ag_recv,
             rs_send_sems, rs_recv_sems, ag_send_sems, ag_recv_sems):
        my = lax.axis_index("i")
        peers = [lax.rem(my + k, N_DEV) for k in range(1, N_DEV)]

        barrier_sem = pltpu.get_barrier_semaphore()
        for p in peers:
            pl.semaphore_signal(
                barrier_sem, inc=1,
                device_id=(p,), device_id_type=pl.DeviceIdType.MESH,
            )
        pl.semaphore_wait(barrier_sem, N_DEV - 1)

        gate = jnp.dot(x_ref[:, :], wg_ref[:, :], preferred_element_type=jnp.float32)
        up = jnp.dot(x_ref[:, :], wu_ref[:, :], preferred_element_type=jnp.float32)
        hidden = gate * (up * jax.nn.sigmoid(up))
        partial = jnp.dot(hidden, wd_ref[:, :], preferred_element_type=jnp.float32)
        out_ref[:, :] = partial
        rs_send[:, :] = partial.astype(jnp.bfloat16)

        def rs_desc(p):
            return pltpu.make_async_remote_copy(
                src_ref=rs_send.at[pl.ds(p * B, B), :],
                dst_ref=rs_recv.at[my],
                send_sem=rs_send_sems.at[p],
                recv_sem=rs_recv_sems.at[my],
                device_id=(p,),
                device_id_type=pl.DeviceIdType.MESH,
            )

        def rs_wait_desc(q):
            return pltpu.make_async_remote_copy(
                src_ref=rs_recv.at[q],
                dst_ref=rs_recv.at[q],
                send_sem=rs_recv_sems.at[q],
                recv_sem=rs_recv_sems.at[q],
                device_id=(q,),
                device_id_type=pl.DeviceIdType.MESH,
            )

        for p in peers:
            rs_desc(p).start()

        acc = out_ref[pl.ds(my * B, B), :]
        for q in peers:
            rs_wait_desc(q).wait_recv()
            acc = acc + rs_recv[q].astype(jnp.float32)
        out_ref[pl.ds(my * B, B), :] = acc
        ag_send[:, :] = acc.astype(jnp.bfloat16)

        def ag_desc(p):
            return pltpu.make_async_remote_copy(
                src_ref=ag_send,
                dst_ref=ag_recv.at[my],
                send_sem=ag_send_sems.at[p],
                recv_sem=ag_recv_sems.at[my],
                device_id=(p,),
                device_id_type=pl.DeviceIdType.MESH,
            )

        def ag_wait_desc(q):
            return pltpu.make_async_remote_copy(
                src_ref=ag_recv.at[q],
                dst_ref=ag_recv.at[q],
                send_sem=ag_recv_sems.at[q],
                recv_sem=ag_recv_sems.at[q],
                device_id=(q,),
                device_id_type=pl.DeviceIdType.MESH,
            )

        for p in peers:
            ag_desc(p).start()

        for q in peers:
            ag_wait_desc(q).wait_recv()
            out_ref[pl.ds(q * B, B), :] = ag_recv[q].astype(jnp.float32)

        for p in peers:
            rs_desc(p).wait_send()
            ag_desc(p).wait_send()

    return pl.pallas_call(
        body,
        out_shape=jax.ShapeDtypeStruct((m, d), jnp.float32),
        in_specs=[pl.BlockSpec(memory_space=pltpu.VMEM)] * 4,
        out_specs=pl.BlockSpec(memory_space=pltpu.VMEM),
        scratch_shapes=[
            pltpu.VMEM((m, d), jnp.bfloat16),
            pltpu.VMEM((N_DEV, B, d), jnp.bfloat16),
            pltpu.VMEM((B, d), jnp.bfloat16),
            pltpu.VMEM((N_DEV, B, d), jnp.bfloat16),
            pltpu.SemaphoreType.DMA((N_DEV,)),
            pltpu.SemaphoreType.DMA((N_DEV,)),
            pltpu.SemaphoreType.DMA((N_DEV,)),
            pltpu.SemaphoreType.DMA((N_DEV,)),
        ],
        compiler_params=pltpu.CompilerParams(collective_id=0),
    )(x, Wg, Wu, Wd)


# device time: 35016 ns/iter; 6.0397x vs baseline; 1.1494x over previous
import jax
import jax.numpy as jnp
from jax import lax
from jax.experimental import pallas as pl
from jax.experimental.pallas import tpu as pltpu

N_DEV = 8


def kernel(x, Wg, Wu, Wd):
    m, d = x.shape
    B = m // N_DEV

    def body(x_ref, wg_ref, wu_ref, wd_ref, out_ref,
             rs_send, rs_recv, ag_send, ag_recv,
             rs_send_sems, rs_recv_sems, ag_send_sems, ag_recv_sems):
        my = lax.axis_index("i")
        peers = [lax.rem(my + k, N_DEV) for k in range(1, N_DEV)]

        barrier_sem = pltpu.get_barrier_semaphore()
        for p in peers:
            pl.semaphore_signal(
                barrier_sem, inc=1,
                device_id=(p,), device_id_type=pl.DeviceIdType.MESH,
            )
        pl.semaphore_wait(barrier_sem, N_DEV - 1)

        def rs_desc(p):
            return pltpu.make_async_remote_copy(
                src_ref=rs_send.at[pl.ds(p * B, B), :],
                dst_ref=rs_recv.at[my],
                send_sem=rs_send_sems.at[p],
                recv_sem=rs_recv_sems.at[my],
                device_id=(p,),
                device_id_type=pl.DeviceIdType.MESH,
            )

        def rs_wait_desc(q):
            return pltpu.make_async_remote_copy(
                src_ref=rs_recv.at[q],
                dst_ref=rs_recv.at[q],
                send_sem=rs_recv_sems.at[q],
                recv_sem=rs_recv_sems.at[q],
                device_id=(q,),
                device_id_type=pl.DeviceIdType.MESH,
            )

        R = 2 * B
        NC = m // R
        my_chunk = my >> 1
        for j in range(1, NC + 1):
            c = lax.rem(my_chunk + j, NC)
            rows = pl.ds(c * R, R)
            xs = x_ref[rows, :]
            gate = jnp.dot(xs, wg_ref[:, :], preferred_element_type=jnp.float32)
            up = jnp.dot(xs, wu_ref[:, :], preferred_element_type=jnp.float32)
            hidden = gate * (up * jax.nn.sigmoid(up))
            partial_c = jnp.dot(hidden, wd_ref[:, :], preferred_element_type=jnp.float32)
            out_ref[rows, :] = partial_c
            rs_send[rows, :] = partial_c.astype(jnp.bfloat16)
            for t in range(2):
                b = 2 * c + t

                @pl.when(b != my)
                def _():
                    rs_desc(b).start()

        acc = out_ref[pl.ds(my * B, B), :]
        for q in peers:
            rs_wait_desc(q).wait_recv()
            acc = acc + rs_recv[q].astype(jnp.float32)
        out_ref[pl.ds(my * B, B), :] = acc
        ag_send[:, :] = acc.astype(jnp.bfloat16)

        def ag_desc(p):
            return pltpu.make_async_remote_copy(
                src_ref=ag_send,
                dst_ref=ag_recv.at[my],
                send_sem=ag_send_sems.at[p],
                recv_sem=ag_recv_sems.at[my],
                device_id=(p,),
                device_id_type=pl.DeviceIdType.MESH,
            )

        def ag_wait_desc(q):
            return pltpu.make_async_remote_copy(
                src_ref=ag_recv.at[q],
                dst_ref=ag_recv.at[q],
                send_sem=ag_recv_sems.at[q],
                recv_sem=ag_recv_sems.at[q],
                device_id=(q,),
                device_id_type=pl.DeviceIdType.MESH,
            )

        for p in peers:
            ag_desc(p).start()

        for q in peers:
            ag_wait_desc(q).wait_recv()
            out_ref[pl.ds(q * B, B), :] = ag_recv[q].astype(jnp.float32)

        for p in peers:
            rs_desc(p).wait_send()
            ag_desc(p).wait_send()

    return pl.pallas_call(
        body,
        out_shape=jax.ShapeDtypeStruct((m, d), jnp.float32),
        in_specs=[pl.BlockSpec(memory_space=pltpu.VMEM)] * 4,
        out_specs=pl.BlockSpec(memory_space=pltpu.VMEM),
        scratch_shapes=[
            pltpu.VMEM((m, d), jnp.bfloat16),
            pltpu.VMEM((N_DEV, B, d), jnp.bfloat16),
            pltpu.VMEM((B, d), jnp.bfloat16),
            pltpu.VMEM((N_DEV, B, d), jnp.bfloat16),
            pltpu.SemaphoreType.DMA((N_DEV,)),
            pltpu.SemaphoreType.DMA((N_DEV,)),
            pltpu.SemaphoreType.DMA((N_DEV,)),
            pltpu.SemaphoreType.DMA((N_DEV,)),
        ],
        compiler_params=pltpu.CompilerParams(collective_id=0),
    )(x, Wg, Wu, Wd)
